# baseline (speedup 1.0000x reference)
import functools

import jax
import jax.numpy as jnp
from jax import lax
from jax.experimental import pallas as pl
from jax.experimental.pallas import tpu as pltpu
from jax.experimental.pallas import tpu_sc as plsc

NC, NS = 2, 16
NW = NC * NS
CHUNK = 128
BN = 512
MLOG = BN.bit_length() - 1


def _tc_relayout(tableT, vocab, embed):
    grid = (vocab + BN - 1) // BN
    rows = grid * (BN // 2)

    def body(x_ref, o_ref):
        x = x_ref[...]
        o_ref[...] = jnp.concatenate([x[:, : BN // 2].T, x[:, BN // 2 :].T], axis=1)

    return pl.pallas_call(
        body,
        grid=(grid,),
        in_specs=[pl.BlockSpec((embed, BN), lambda i: (0, i))],
        out_specs=pl.BlockSpec((BN // 2, 2 * embed), lambda i: (i, 0)),
        out_shape=jax.ShapeDtypeStruct((rows, 2 * embed), jnp.float32),
    )(tableT)


def _sc_gather(table2, idx3, n_chunks, width):
    rows_per_w = n_chunks * CHUNK
    half = rows_per_w // 2
    total = NW * rows_per_w
    mesh = plsc.VectorSubcoreMesh(core_axis_name="c", subcore_axis_name="s")

    @functools.partial(
        pl.kernel,
        out_type=jax.ShapeDtypeStruct((total, width), jnp.float32),
        mesh=mesh,
        scratch_types=[
            pltpu.VMEM((n_chunks, CHUNK), jnp.int32),
            pltpu.VMEM((half, width), jnp.float32),
            pltpu.SemaphoreType.DMA,
        ],
        compiler_params=pltpu.CompilerParams(use_tc_tiling_on_sc=True),
    )
    def gather_kernel(table_hbm, idx_hbm, out_hbm, idx_v, rows_v, sem):
        wid = lax.axis_index("s") * NC + lax.axis_index("c")
        pltpu.sync_copy(idx_hbm.at[wid], idx_v)
        for h in range(2):
            copies = [
                pltpu.async_copy(
                    table_hbm.at[idx_v.at[h * (n_chunks // 2) + j]],
                    rows_v.at[pl.ds(j * CHUNK, CHUNK)],
                    sem,
                )
                for j in range(n_chunks // 2)
            ]
            for c in copies:
                c.wait()
            pltpu.sync_copy(rows_v, out_hbm.at[pl.ds(wid * rows_per_w + h * half, half)])

    return gather_kernel(table2, idx3)


def kernel(current_node, actionList, table, W, b):
    B = current_node.shape[0]
    vocab, embed = table.shape
    width = 2 * embed
    last_node = actionList[:, -2]
    idx = jnp.concatenate([current_node, last_node]).astype(jnp.int32)
    pair_row = ((idx >> MLOG) << (MLOG - 1)) | (idx & (BN // 2 - 1))
    half_flag = ((idx >> (MLOG - 1)) & 1).reshape(2 * B, 1)

    n_chunks = (2 * B) // (NW * CHUNK)
    idx3 = pair_row.reshape(NW, n_chunks, CHUNK)

    table2 = _tc_relayout(table.T, vocab, embed)
    gathered = _sc_gather(table2, idx3, n_chunks, width)

    BM = 2048
    grid = B // BM
    w1 = W[:embed]
    w2 = W[embed:]
    b2 = b.reshape(1, embed)

    def proj(cur_ref, last_ref, hcur_ref, hlast_ref, w1_ref, w2_ref, b_ref, o_ref):
        cur_pair = cur_ref[...]
        last_pair = last_ref[...]
        cur_e = jnp.where(hcur_ref[...] == 0, cur_pair[:, :embed], cur_pair[:, embed:])
        last_e = jnp.where(hlast_ref[...] == 0, last_pair[:, :embed], last_pair[:, embed:])
        o_ref[...] = (
            jnp.dot(cur_e, w1_ref[...], preferred_element_type=jnp.float32)
            + jnp.dot(last_e, w2_ref[...], preferred_element_type=jnp.float32)
            + b_ref[...]
        )

    return pl.pallas_call(
        proj,
        grid=(grid,),
        in_specs=[
            pl.BlockSpec((BM, width), lambda i: (i, 0)),
            pl.BlockSpec((BM, width), lambda i: (i + grid, 0)),
            pl.BlockSpec((BM, 1), lambda i: (i, 0)),
            pl.BlockSpec((BM, 1), lambda i: (i + grid, 0)),
            pl.BlockSpec((embed, embed), lambda i: (0, 0)),
            pl.BlockSpec((embed, embed), lambda i: (0, 0)),
            pl.BlockSpec((1, embed), lambda i: (0, 0)),
        ],
        out_specs=pl.BlockSpec((BM, embed), lambda i: (i, 0)),
        out_shape=jax.ShapeDtypeStruct((B, embed), jnp.float32),
    )(gathered, gathered, half_flag, half_flag, w1, w2, b2)


# speedup vs baseline: 3.7991x; 3.7991x over previous
import functools

import jax
import jax.numpy as jnp
from jax import lax
from jax.experimental import pallas as pl
from jax.experimental.pallas import tpu as pltpu
from jax.experimental.pallas import tpu_sc as plsc

NC, NS = 2, 16
NW = NC * NS
CHUNK = 128
BN = 4096
MLOG = BN.bit_length() - 1


def _tc_relayout(tableT, vocab, embed):
    grid = (vocab + BN - 1) // BN
    rows = grid * (BN // 2)

    def body(x_ref, o_ref):
        x = x_ref[...]
        o_ref[...] = jnp.concatenate([x[:, : BN // 2], x[:, BN // 2 :]], axis=0).T

    return pl.pallas_call(
        body,
        grid=(grid,),
        in_specs=[pl.BlockSpec((embed, BN), lambda i: (0, i))],
        out_specs=pl.BlockSpec((BN // 2, 2 * embed), lambda i: (i, 0)),
        out_shape=jax.ShapeDtypeStruct((rows, 2 * embed), jnp.float32),
    )(tableT)


def _sc_gather(table2, idx3, n_chunks, width):
    rows_per_w = n_chunks * CHUNK
    half = rows_per_w // 2
    total = NW * rows_per_w
    mesh = plsc.VectorSubcoreMesh(core_axis_name="c", subcore_axis_name="s")

    @functools.partial(
        pl.kernel,
        out_type=jax.ShapeDtypeStruct((total, width), jnp.float32),
        mesh=mesh,
        scratch_types=[
            pltpu.VMEM((n_chunks, CHUNK), jnp.int32),
            pltpu.VMEM((half, width), jnp.float32),
            pltpu.SemaphoreType.DMA,
        ],
        compiler_params=pltpu.CompilerParams(use_tc_tiling_on_sc=True),
    )
    def gather_kernel(table_hbm, idx_hbm, out_hbm, idx_v, rows_v, sem):
        wid = lax.axis_index("s") * NC + lax.axis_index("c")
        pltpu.sync_copy(idx_hbm.at[wid], idx_v)
        for h in range(2):
            copies = [
                pltpu.async_copy(
                    table_hbm.at[idx_v.at[h * (n_chunks // 2) + j]],
                    rows_v.at[pl.ds(j * CHUNK, CHUNK)],
                    sem,
                )
                for j in range(n_chunks // 2)
            ]
            for c in copies:
                c.wait()
            pltpu.sync_copy(rows_v, out_hbm.at[pl.ds(wid * rows_per_w + h * half, half)])

    return gather_kernel(table2, idx3)


def kernel(current_node, actionList, table, W, b):
    B = current_node.shape[0]
    vocab, embed = table.shape
    width = 2 * embed
    last_node = actionList[:, -2]
    idx = jnp.concatenate([current_node, last_node]).astype(jnp.int32)
    pair_row = ((idx >> MLOG) << (MLOG - 1)) | (idx & (BN // 2 - 1))
    half_flag = ((idx >> (MLOG - 1)) & 1).reshape(2 * B, 1)

    n_chunks = (2 * B) // (NW * CHUNK)
    idx3 = pair_row.reshape(NW, n_chunks, CHUNK)

    table2 = _tc_relayout(table.T, vocab, embed)
    gathered = _sc_gather(table2, idx3, n_chunks, width)

    BM = 2048
    grid = B // BM
    w1 = W[:embed]
    w2 = W[embed:]
    b2 = b.reshape(1, embed)

    def proj(cur_ref, last_ref, hcur_ref, hlast_ref, w1_ref, w2_ref, b_ref, o_ref):
        cur_pair = cur_ref[...]
        last_pair = last_ref[...]
        cur_e = jnp.where(hcur_ref[...] == 0, cur_pair[:, :embed], cur_pair[:, embed:])
        last_e = jnp.where(hlast_ref[...] == 0, last_pair[:, :embed], last_pair[:, embed:])
        o_ref[...] = (
            jnp.dot(cur_e, w1_ref[...], preferred_element_type=jnp.float32)
            + jnp.dot(last_e, w2_ref[...], preferred_element_type=jnp.float32)
            + b_ref[...]
        )

    return pl.pallas_call(
        proj,
        grid=(grid,),
        in_specs=[
            pl.BlockSpec((BM, width), lambda i: (i, 0)),
            pl.BlockSpec((BM, width), lambda i: (i + grid, 0)),
            pl.BlockSpec((BM, 1), lambda i: (i, 0)),
            pl.BlockSpec((BM, 1), lambda i: (i + grid, 0)),
            pl.BlockSpec((embed, embed), lambda i: (0, 0)),
            pl.BlockSpec((embed, embed), lambda i: (0, 0)),
            pl.BlockSpec((1, embed), lambda i: (0, 0)),
        ],
        out_specs=pl.BlockSpec((BM, embed), lambda i: (i, 0)),
        out_shape=jax.ShapeDtypeStruct((B, embed), jnp.float32),
    )(gathered, gathered, half_flag, half_flag, w1, w2, b2)


# speedup vs baseline: 3.8586x; 1.0156x over previous
import functools

import jax
import jax.numpy as jnp
from jax import lax
from jax.experimental import pallas as pl
from jax.experimental.pallas import tpu as pltpu
from jax.experimental.pallas import tpu_sc as plsc

NC, NS = 2, 16
NW = NC * NS
CHUNK = 128
BN = 4096
MLOG = BN.bit_length() - 1


def _tc_relayout(tableT, vocab, embed):
    grid = (vocab + BN - 1) // BN
    rows = grid * (BN // 2)

    def body(x_ref, o_ref):
        x = x_ref[...]
        o_ref[...] = jnp.concatenate([x[:, : BN // 2], x[:, BN // 2 :]], axis=0).T

    return pl.pallas_call(
        body,
        grid=(grid,),
        in_specs=[pl.BlockSpec((embed, BN), lambda i: (0, i))],
        out_specs=pl.BlockSpec((BN // 2, 2 * embed), lambda i: (i, 0)),
        out_shape=jax.ShapeDtypeStruct((rows, 2 * embed), jnp.float32),
    )(tableT)


def _sc_gather(table2, idx3, n_chunks, width):
    rows_per_w = n_chunks * CHUNK
    half = rows_per_w // 2
    total = NW * rows_per_w
    mesh = plsc.VectorSubcoreMesh(core_axis_name="c", subcore_axis_name="s")

    @functools.partial(
        pl.kernel,
        out_type=jax.ShapeDtypeStruct((total, width), jnp.float32),
        mesh=mesh,
        scratch_types=[
            pltpu.VMEM((n_chunks, CHUNK), jnp.int32),
            pltpu.VMEM((half, width), jnp.float32),
            pltpu.SemaphoreType.DMA,
        ],
        compiler_params=pltpu.CompilerParams(use_tc_tiling_on_sc=True),
    )
    def gather_kernel(table_hbm, idx_hbm, out_hbm, idx_v, rows_v, sem):
        wid = lax.axis_index("s") * NC + lax.axis_index("c")
        pltpu.sync_copy(idx_hbm.at[wid], idx_v)
        for h in range(2):
            copies = [
                pltpu.async_copy(
                    table_hbm.at[idx_v.at[h * (n_chunks // 2) + j]],
                    rows_v.at[pl.ds(j * CHUNK, CHUNK)],
                    sem,
                )
                for j in range(n_chunks // 2)
            ]
            for c in copies:
                c.wait()
            pltpu.sync_copy(rows_v, out_hbm.at[pl.ds(wid * rows_per_w + h * half, half)])

    return gather_kernel(table2, idx3)


def kernel(current_node, actionList, table, W, b):
    B = current_node.shape[0]
    vocab, embed = table.shape
    width = 2 * embed
    last_node = actionList[:, -2]
    idx = jnp.concatenate([current_node, last_node]).astype(jnp.int32)
    pair_row = ((idx >> MLOG) << (MLOG - 1)) | (idx & (BN // 2 - 1))
    half_flag = ((idx >> (MLOG - 1)) & 1).reshape(2 * B, 1)

    n_chunks = (2 * B) // (NW * CHUNK)
    idx3 = pair_row.reshape(NW, n_chunks, CHUNK)

    table2 = _tc_relayout(table.T, vocab, embed)
    gathered = _sc_gather(table2, idx3, n_chunks, width)

    BM = 2048
    grid = B // BM
    w1 = W[:embed]
    w2 = W[embed:]
    b2 = b.reshape(1, embed)

    def proj(cur_ref, last_ref, hcur_ref, hlast_ref, w1_ref, w2_ref, b_ref, o_ref):
        cur_pair = cur_ref[...]
        last_pair = last_ref[...]
        cur_e = jnp.where(hcur_ref[...] == 0, cur_pair[:, :embed], cur_pair[:, embed:])
        last_e = jnp.where(hlast_ref[...] == 0, last_pair[:, :embed], last_pair[:, embed:])
        res = (
            jnp.dot(cur_e, w1_ref[...], preferred_element_type=jnp.float32)
            + jnp.dot(last_e, w2_ref[...], preferred_element_type=jnp.float32)
            + b_ref[...]
        )
        o_ref[...] = res.T

    outT = pl.pallas_call(
        proj,
        grid=(grid,),
        in_specs=[
            pl.BlockSpec((BM, width), lambda i: (i, 0)),
            pl.BlockSpec((BM, width), lambda i: (i + grid, 0)),
            pl.BlockSpec((BM, 1), lambda i: (i, 0)),
            pl.BlockSpec((BM, 1), lambda i: (i + grid, 0)),
            pl.BlockSpec((embed, embed), lambda i: (0, 0)),
            pl.BlockSpec((embed, embed), lambda i: (0, 0)),
            pl.BlockSpec((1, embed), lambda i: (0, 0)),
        ],
        out_specs=pl.BlockSpec((embed, BM), lambda i: (0, i)),
        out_shape=jax.ShapeDtypeStruct((embed, B), jnp.float32),
    )(gathered, gathered, half_flag, half_flag, w1, w2, b2)
    return outT.T


# speedup vs baseline: 4.0887x; 1.0597x over previous
import functools

import jax
import jax.numpy as jnp
from jax import lax
from jax.experimental import pallas as pl
from jax.experimental.pallas import tpu as pltpu
from jax.experimental.pallas import tpu_sc as plsc

NC, NS = 2, 16
NW = NC * NS
CHUNK = 128
BN = 4096
MLOG = BN.bit_length() - 1
Q = BN // 4


def _tc_relayout(tableT, vocab, embed):
    grid = (vocab + BN - 1) // BN
    rows = grid * Q

    def body(x_ref, o_ref):
        x = x_ref[...]
        xs = jnp.concatenate(
            [x[:, :Q], x[:, Q : 2 * Q], x[:, 2 * Q : 3 * Q], x[:, 3 * Q :]], axis=0
        )
        y = xs.T.astype(jnp.bfloat16)
        lo = lax.bitcast_convert_type(y[:, : 2 * embed], jnp.uint16).astype(jnp.uint32)
        hi = lax.bitcast_convert_type(y[:, 2 * embed :], jnp.uint16).astype(jnp.uint32)
        o_ref[...] = ((hi << 16) | lo).astype(jnp.int32)

    return pl.pallas_call(
        body,
        grid=(grid,),
        in_specs=[pl.BlockSpec((embed, BN), lambda i: (0, i))],
        out_specs=pl.BlockSpec((Q, 2 * embed), lambda i: (i, 0)),
        out_shape=jax.ShapeDtypeStruct((rows, 2 * embed), jnp.int32),
    )(tableT)


def _sc_gather(table3, idx3, n_chunks, width):
    rows_per_w = n_chunks * CHUNK
    half = rows_per_w // 2
    total = NW * rows_per_w
    mesh = plsc.VectorSubcoreMesh(core_axis_name="c", subcore_axis_name="s")

    @functools.partial(
        pl.kernel,
        out_type=jax.ShapeDtypeStruct((total, width), jnp.int32),
        mesh=mesh,
        scratch_types=[
            pltpu.VMEM((n_chunks, CHUNK), jnp.int32),
            pltpu.VMEM((half, width), jnp.int32),
            pltpu.SemaphoreType.DMA,
        ],
        compiler_params=pltpu.CompilerParams(use_tc_tiling_on_sc=True),
    )
    def gather_kernel(table_hbm, idx_hbm, out_hbm, idx_v, rows_v, sem):
        wid = lax.axis_index("s") * NC + lax.axis_index("c")
        pltpu.sync_copy(idx_hbm.at[wid], idx_v)
        for h in range(2):
            copies = [
                pltpu.async_copy(
                    table_hbm.at[idx_v.at[h * (n_chunks // 2) + j]],
                    rows_v.at[pl.ds(j * CHUNK, CHUNK)],
                    sem,
                )
                for j in range(n_chunks // 2)
            ]
            for c in copies:
                c.wait()
            pltpu.sync_copy(rows_v, out_hbm.at[pl.ds(wid * rows_per_w + h * half, half)])

    return gather_kernel(table3, idx3)


def kernel(current_node, actionList, table, W, b):
    B = current_node.shape[0]
    vocab, embed = table.shape
    width = 2 * embed
    last_node = actionList[:, -2]
    idx = jnp.concatenate([current_node, last_node]).astype(jnp.int32)
    quad_row = ((idx >> MLOG) << (MLOG - 2)) | (idx & (Q - 1))
    sl_flag = ((idx >> (MLOG - 1)) & 1).reshape(2 * B, 1)
    half_flag = ((idx >> (MLOG - 2)) & 1).reshape(2 * B, 1)

    n_chunks = (2 * B) // (NW * CHUNK)
    idx3 = quad_row.reshape(NW, n_chunks, CHUNK)

    table3 = _tc_relayout(table.T, vocab, embed)
    gathered = _sc_gather(table3, idx3, n_chunks, width)

    BM = 2048
    grid = B // BM
    w1 = W[:embed]
    w2 = W[embed:]
    b2 = b.reshape(1, embed)

    def pick(g, sl, hf):
        word = jnp.where(sl == 0, g & 0xFFFF, (g >> 16) & 0xFFFF)
        row = lax.bitcast_convert_type(word.astype(jnp.uint16), jnp.bfloat16)
        return jnp.where(hf == 0, row[:, :embed], row[:, embed:]).astype(jnp.float32)

    def proj(cur_ref, last_ref, scur_ref, slast_ref, hcur_ref, hlast_ref,
             w1_ref, w2_ref, b_ref, o_ref):
        cur_e = pick(cur_ref[...], scur_ref[...], hcur_ref[...])
        last_e = pick(last_ref[...], slast_ref[...], hlast_ref[...])
        res = (
            jnp.dot(cur_e, w1_ref[...], preferred_element_type=jnp.float32)
            + jnp.dot(last_e, w2_ref[...], preferred_element_type=jnp.float32)
            + b_ref[...]
        )
        o_ref[...] = res.T

    outT = pl.pallas_call(
        proj,
        grid=(grid,),
        in_specs=[
            pl.BlockSpec((BM, width), lambda i: (i, 0)),
            pl.BlockSpec((BM, width), lambda i: (i + grid, 0)),
            pl.BlockSpec((BM, 1), lambda i: (i, 0)),
            pl.BlockSpec((BM, 1), lambda i: (i + grid, 0)),
            pl.BlockSpec((BM, 1), lambda i: (i, 0)),
            pl.BlockSpec((BM, 1), lambda i: (i + grid, 0)),
            pl.BlockSpec((embed, embed), lambda i: (0, 0)),
            pl.BlockSpec((embed, embed), lambda i: (0, 0)),
            pl.BlockSpec((1, embed), lambda i: (0, 0)),
        ],
        out_specs=pl.BlockSpec((embed, BM), lambda i: (0, i)),
        out_shape=jax.ShapeDtypeStruct((embed, B), jnp.float32),
    )(gathered, gathered, sl_flag, sl_flag, half_flag, half_flag, w1, w2, b2)
    return outT.T


# speedup vs baseline: 4.3756x; 1.0702x over previous
import functools

import jax
import jax.numpy as jnp
from jax import lax
from jax.experimental import pallas as pl
from jax.experimental.pallas import tpu as pltpu
from jax.experimental.pallas import tpu_sc as plsc

NC, NS = 2, 16
NW = NC * NS
CHUNK = 128
BN = 4096
MLOG = BN.bit_length() - 1
Q = BN // 4


def _tc_relayout(tableT, vocab, embed):
    grid = (vocab + BN - 1) // BN
    rows = grid * Q

    def body(x_ref, o_ref):
        x = x_ref[...]
        xs = jnp.concatenate(
            [x[:, :Q], x[:, Q : 2 * Q], x[:, 2 * Q : 3 * Q], x[:, 3 * Q :]], axis=0
        )
        y = xs.T.astype(jnp.bfloat16)
        lo = lax.bitcast_convert_type(y[:, : 2 * embed], jnp.uint16).astype(jnp.uint32)
        hi = lax.bitcast_convert_type(y[:, 2 * embed :], jnp.uint16).astype(jnp.uint32)
        o_ref[...] = ((hi << 16) | lo).astype(jnp.int32)

    return pl.pallas_call(
        body,
        grid=(grid,),
        in_specs=[pl.BlockSpec((embed, BN), lambda i: (0, i))],
        out_specs=pl.BlockSpec((Q, 2 * embed), lambda i: (i, 0)),
        out_shape=jax.ShapeDtypeStruct((rows, 2 * embed), jnp.int32),
    )(tableT)


def _sc_gather(table3, idx3, n_chunks, width):
    rows_per_w = n_chunks * CHUNK
    half = rows_per_w // 2
    total = NW * rows_per_w
    mesh = plsc.VectorSubcoreMesh(core_axis_name="c", subcore_axis_name="s")

    @functools.partial(
        pl.kernel,
        out_type=jax.ShapeDtypeStruct((total, width), jnp.int32),
        mesh=mesh,
        scratch_types=[
            pltpu.VMEM((n_chunks, CHUNK), jnp.int32),
            pltpu.VMEM((half, width), jnp.int32),
            pltpu.SemaphoreType.DMA,
        ],
        compiler_params=pltpu.CompilerParams(use_tc_tiling_on_sc=True),
    )
    def gather_kernel(table_hbm, idx_hbm, out_hbm, idx_v, rows_v, sem):
        wid = lax.axis_index("s") * NC + lax.axis_index("c")
        pltpu.sync_copy(idx_hbm.at[wid], idx_v)
        for h in range(2):
            copies = [
                pltpu.async_copy(
                    table_hbm.at[idx_v.at[h * (n_chunks // 2) + j]],
                    rows_v.at[pl.ds(j * CHUNK, CHUNK)],
                    sem,
                )
                for j in range(n_chunks // 2)
            ]
            for c in copies:
                c.wait()
            pltpu.sync_copy(rows_v, out_hbm.at[pl.ds(wid * rows_per_w + h * half, half)])

    return gather_kernel(table3, idx3)


def kernel(current_node, actionList, table, W, b):
    B = current_node.shape[0]
    vocab, embed = table.shape
    width = 2 * embed
    last_node = actionList[:, -2]
    idx = jnp.concatenate([current_node, last_node]).astype(jnp.int32)
    quad_row = ((idx >> MLOG) << (MLOG - 2)) | (idx & (Q - 1))
    idx2 = idx.reshape(2 * B, 1)

    n_chunks = (2 * B) // (NW * CHUNK)
    idx3 = quad_row.reshape(NW, n_chunks, CHUNK)

    table3 = _tc_relayout(table.T, vocab, embed)
    gathered = _sc_gather(table3, idx3, n_chunks, width)

    BM = 2048
    grid = B // BM
    w1 = W[:embed]
    w2 = W[embed:]
    b2 = b.reshape(1, embed)

    def pick(g, sl, hf):
        word = jnp.where(sl == 0, g & 0xFFFF, (g >> 16) & 0xFFFF)
        row = lax.bitcast_convert_type(word.astype(jnp.uint16), jnp.bfloat16)
        return jnp.where(hf == 0, row[:, :embed], row[:, embed:]).astype(jnp.float32)

    def proj(cur_ref, last_ref, scur_ref, slast_ref, hcur_ref, hlast_ref,
             w1_ref, w2_ref, b_ref, o_ref):
        cur_e = pick(cur_ref[...], scur_ref[...], hcur_ref[...])
        last_e = pick(last_ref[...], slast_ref[...], hlast_ref[...])
        res = (
            jnp.dot(cur_e, w1_ref[...], preferred_element_type=jnp.float32)
            + jnp.dot(last_e, w2_ref[...], preferred_element_type=jnp.float32)
            + b_ref[...]
        )
        o_ref[...] = res.T

    outT = pl.pallas_call(
        proj,
        grid=(grid,),
        in_specs=[
            pl.BlockSpec((BM, width), lambda i: (i, 0)),
            pl.BlockSpec((BM, width), lambda i: (i + grid, 0)),
            pl.BlockSpec((BM, 1), lambda i: (i, 0)),
            pl.BlockSpec((BM, 1), lambda i: (i + grid, 0)),
            pl.BlockSpec((embed, embed), lambda i: (0, 0)),
            pl.BlockSpec((embed, embed), lambda i: (0, 0)),
            pl.BlockSpec((1, embed), lambda i: (0, 0)),
        ],
        out_specs=pl.BlockSpec((embed, BM), lambda i: (0, i)),
        out_shape=jax.ShapeDtypeStruct((embed, B), jnp.float32),
    )(gathered, gathered, idx2, idx2, w1, w2, b2)
    return outT.T


# speedup vs baseline: 5.9196x; 1.3529x over previous
import functools

import jax
import jax.numpy as jnp
from jax import lax
from jax.experimental import pallas as pl
from jax.experimental.pallas import tpu as pltpu
from jax.experimental.pallas import tpu_sc as plsc

NC, NS = 2, 16
NW = NC * NS
CHUNK = 128
BN = 8192
MLOG = BN.bit_length() - 1
Q = BN // 4


def _tc_relayout(tableT, vocab, embed):
    grid = (vocab + BN - 1) // BN
    rows = grid * Q

    def body(x_ref, o_ref):
        x = x_ref[...]
        xs = jnp.concatenate(
            [x[:, :Q], x[:, Q : 2 * Q], x[:, 2 * Q : 3 * Q], x[:, 3 * Q :]], axis=0
        )
        y = xs.T.astype(jnp.bfloat16)
        lo = lax.bitcast_convert_type(y[:, : 2 * embed], jnp.uint16).astype(jnp.uint32)
        hi = lax.bitcast_convert_type(y[:, 2 * embed :], jnp.uint16).astype(jnp.uint32)
        o_ref[...] = ((hi << 16) | lo).astype(jnp.int32)

    return pl.pallas_call(
        body,
        grid=(grid,),
        in_specs=[pl.BlockSpec((embed, BN), lambda i: (0, i))],
        out_specs=pl.BlockSpec((Q, 2 * embed), lambda i: (i, 0)),
        out_shape=jax.ShapeDtypeStruct((rows, 2 * embed), jnp.int32),
    )(tableT)


def _sc_gather(table3, idx3, n_chunks, width):
    rows_per_w = n_chunks * CHUNK
    half = rows_per_w // 2
    total = NW * rows_per_w
    mesh = plsc.VectorSubcoreMesh(core_axis_name="c", subcore_axis_name="s")

    @functools.partial(
        pl.kernel,
        out_type=jax.ShapeDtypeStruct((total, width), jnp.int32),
        mesh=mesh,
        scratch_types=[
            pltpu.VMEM((n_chunks, CHUNK), jnp.int32),
            pltpu.VMEM((half, width), jnp.int32),
            pltpu.SemaphoreType.DMA,
        ],
        compiler_params=pltpu.CompilerParams(use_tc_tiling_on_sc=True),
    )
    def gather_kernel(table_hbm, idx_hbm, out_hbm, idx_v, rows_v, sem):
        wid = lax.axis_index("s") * NC + lax.axis_index("c")
        pltpu.sync_copy(idx_hbm.at[wid], idx_v)
        for h in range(2):
            copies = [
                pltpu.async_copy(
                    table_hbm.at[idx_v.at[h * (n_chunks // 2) + j]],
                    rows_v.at[pl.ds(j * CHUNK, CHUNK)],
                    sem,
                )
                for j in range(n_chunks // 2)
            ]
            for c in copies:
                c.wait()
            pltpu.sync_copy(rows_v, out_hbm.at[pl.ds(wid * rows_per_w + h * half, half)])

    return gather_kernel(table3, idx3)


def kernel(current_node, actionList, table, W, b):
    B = current_node.shape[0]
    vocab, embed = table.shape
    width = 2 * embed
    last_node = actionList[:, -2]
    idx = jnp.concatenate([current_node, last_node]).astype(jnp.int32)
    quad_row = ((idx >> MLOG) << (MLOG - 2)) | (idx & (Q - 1))
    idx2 = idx.reshape(2 * B, 1)

    n_chunks = (2 * B) // (NW * CHUNK)
    idx3 = quad_row.reshape(NW, n_chunks, CHUNK)

    table3 = _tc_relayout(table.T, vocab, embed)
    gathered = _sc_gather(table3, idx3, n_chunks, width)

    BM = 2048
    grid = B // BM
    w1 = W[:embed]
    w2 = W[embed:]
    b2 = b.reshape(1, embed)

    def pick(g, sl, hf):
        word = jnp.where(sl == 0, g & 0xFFFF, (g >> 16) & 0xFFFF)
        row = lax.bitcast_convert_type(word.astype(jnp.uint16), jnp.bfloat16)
        return jnp.where(hf == 0, row[:, :embed], row[:, embed:]).astype(jnp.float32)

    def proj(cur_ref, last_ref, scur_ref, slast_ref, hcur_ref, hlast_ref,
             w1_ref, w2_ref, b_ref, o_ref):
        cur_e = pick(cur_ref[...], scur_ref[...], hcur_ref[...])
        last_e = pick(last_ref[...], slast_ref[...], hlast_ref[...])
        res = (
            jnp.dot(cur_e, w1_ref[...], preferred_element_type=jnp.float32)
            + jnp.dot(last_e, w2_ref[...], preferred_element_type=jnp.float32)
            + b_ref[...]
        )
        o_ref[...] = res.T

    outT = pl.pallas_call(
        proj,
        grid=(grid,),
        in_specs=[
            pl.BlockSpec((BM, width), lambda i: (i, 0)),
            pl.BlockSpec((BM, width), lambda i: (i + grid, 0)),
            pl.BlockSpec((BM, 1), lambda i: (i, 0)),
            pl.BlockSpec((BM, 1), lambda i: (i + grid, 0)),
            pl.BlockSpec((embed, embed), lambda i: (0, 0)),
            pl.BlockSpec((embed, embed), lambda i: (0, 0)),
            pl.BlockSpec((1, embed), lambda i: (0, 0)),
        ],
        out_specs=pl.BlockSpec((embed, BM), lambda i: (0, i)),
        out_shape=jax.ShapeDtypeStruct((embed, B), jnp.float32),
    )(gathered, gathered, idx2, idx2, w1, w2, b2)
    return outT.T


# speedup vs baseline: 7.0376x; 1.1889x over previous
import functools

import jax
import jax.numpy as jnp
from jax import lax
from jax.experimental import pallas as pl
from jax.experimental.pallas import tpu as pltpu
from jax.experimental.pallas import tpu_sc as plsc

NC, NS = 2, 16
NW = NC * NS
CHUNK = 128
BN = 16384
MLOG = BN.bit_length() - 1
Q = BN // 4


def _tc_relayout(tableT, vocab, embed):
    grid = (vocab + BN - 1) // BN
    rows = grid * Q

    def body(x_ref, o_ref):
        x = x_ref[...]
        xs = jnp.concatenate(
            [x[:, :Q], x[:, Q : 2 * Q], x[:, 2 * Q : 3 * Q], x[:, 3 * Q :]], axis=0
        )
        y = xs.T.astype(jnp.bfloat16)
        lo = lax.bitcast_convert_type(y[:, : 2 * embed], jnp.uint16).astype(jnp.uint32)
        hi = lax.bitcast_convert_type(y[:, 2 * embed :], jnp.uint16).astype(jnp.uint32)
        o_ref[...] = ((hi << 16) | lo).astype(jnp.int32)

    return pl.pallas_call(
        body,
        grid=(grid,),
        in_specs=[pl.BlockSpec((embed, BN), lambda i: (0, i))],
        out_specs=pl.BlockSpec((Q, 2 * embed), lambda i: (i, 0)),
        out_shape=jax.ShapeDtypeStruct((rows, 2 * embed), jnp.int32),
    )(tableT)


def _sc_gather(table3, idx3, n_chunks, width):
    rows_per_w = n_chunks * CHUNK
    half = rows_per_w // 2
    total = NW * rows_per_w
    mesh = plsc.VectorSubcoreMesh(core_axis_name="c", subcore_axis_name="s")

    @functools.partial(
        pl.kernel,
        out_type=jax.ShapeDtypeStruct((total, width), jnp.int32),
        mesh=mesh,
        scratch_types=[
            pltpu.VMEM((n_chunks, CHUNK), jnp.int32),
            pltpu.VMEM((half, width), jnp.int32),
            pltpu.SemaphoreType.DMA,
        ],
        compiler_params=pltpu.CompilerParams(use_tc_tiling_on_sc=True),
    )
    def gather_kernel(table_hbm, idx_hbm, out_hbm, idx_v, rows_v, sem):
        wid = lax.axis_index("s") * NC + lax.axis_index("c")
        pltpu.sync_copy(idx_hbm.at[wid], idx_v)
        for h in range(2):
            copies = [
                pltpu.async_copy(
                    table_hbm.at[idx_v.at[h * (n_chunks // 2) + j]],
                    rows_v.at[pl.ds(j * CHUNK, CHUNK)],
                    sem,
                )
                for j in range(n_chunks // 2)
            ]
            for c in copies:
                c.wait()
            pltpu.sync_copy(rows_v, out_hbm.at[pl.ds(wid * rows_per_w + h * half, half)])

    return gather_kernel(table3, idx3)


def kernel(current_node, actionList, table, W, b):
    B = current_node.shape[0]
    vocab, embed = table.shape
    width = 2 * embed
    last_node = actionList[:, -2]
    idx = jnp.concatenate([current_node, last_node]).astype(jnp.int32)
    quad_row = ((idx >> MLOG) << (MLOG - 2)) | (idx & (Q - 1))
    idx2 = idx.reshape(2 * B, 1)

    n_chunks = (2 * B) // (NW * CHUNK)
    idx3 = quad_row.reshape(NW, n_chunks, CHUNK)

    table3 = _tc_relayout(table.T, vocab, embed)
    gathered = _sc_gather(table3, idx3, n_chunks, width)

    BM = 2048
    grid = B // BM
    w1 = W[:embed]
    w2 = W[embed:]
    b2 = b.reshape(1, embed)

    def pick(g, sl, hf):
        word = jnp.where(sl == 0, g & 0xFFFF, (g >> 16) & 0xFFFF)
        row = lax.bitcast_convert_type(word.astype(jnp.uint16), jnp.bfloat16)
        return jnp.where(hf == 0, row[:, :embed], row[:, embed:]).astype(jnp.float32)

    def proj(cur_ref, last_ref, scur_ref, slast_ref, hcur_ref, hlast_ref,
             w1_ref, w2_ref, b_ref, o_ref):
        cur_e = pick(cur_ref[...], scur_ref[...], hcur_ref[...])
        last_e = pick(last_ref[...], slast_ref[...], hlast_ref[...])
        res = (
            jnp.dot(cur_e, w1_ref[...], preferred_element_type=jnp.float32)
            + jnp.dot(last_e, w2_ref[...], preferred_element_type=jnp.float32)
            + b_ref[...]
        )
        o_ref[...] = res.T

    outT = pl.pallas_call(
        proj,
        grid=(grid,),
        in_specs=[
            pl.BlockSpec((BM, width), lambda i: (i, 0)),
            pl.BlockSpec((BM, width), lambda i: (i + grid, 0)),
            pl.BlockSpec((BM, 1), lambda i: (i, 0)),
            pl.BlockSpec((BM, 1), lambda i: (i + grid, 0)),
            pl.BlockSpec((embed, embed), lambda i: (0, 0)),
            pl.BlockSpec((embed, embed), lambda i: (0, 0)),
            pl.BlockSpec((1, embed), lambda i: (0, 0)),
        ],
        out_specs=pl.BlockSpec((embed, BM), lambda i: (0, i)),
        out_shape=jax.ShapeDtypeStruct((embed, B), jnp.float32),
    )(gathered, gathered, idx2, idx2, w1, w2, b2)
    return outT.T


# speedup vs baseline: 7.2679x; 1.0327x over previous
import functools

import jax
import jax.numpy as jnp
from jax import lax
from jax.experimental import pallas as pl
from jax.experimental.pallas import tpu as pltpu
from jax.experimental.pallas import tpu_sc as plsc

NC, NS = 2, 16
NW = NC * NS
CHUNK = 128
BN = 32768
MLOG = BN.bit_length() - 1
Q = BN // 4


def _tc_relayout(tableT, vocab, embed):
    grid = (vocab + BN - 1) // BN
    rows = grid * Q

    def body(x_ref, o_ref):
        x = x_ref[...]
        xs = jnp.concatenate(
            [x[:, :Q], x[:, Q : 2 * Q], x[:, 2 * Q : 3 * Q], x[:, 3 * Q :]], axis=0
        )
        y = xs.T.astype(jnp.bfloat16)
        lo = lax.bitcast_convert_type(y[:, : 2 * embed], jnp.uint16).astype(jnp.uint32)
        hi = lax.bitcast_convert_type(y[:, 2 * embed :], jnp.uint16).astype(jnp.uint32)
        o_ref[...] = ((hi << 16) | lo).astype(jnp.int32)

    return pl.pallas_call(
        body,
        grid=(grid,),
        in_specs=[pl.BlockSpec((embed, BN), lambda i: (0, i))],
        out_specs=pl.BlockSpec((Q, 2 * embed), lambda i: (i, 0)),
        out_shape=jax.ShapeDtypeStruct((rows, 2 * embed), jnp.int32),
    )(tableT)


def _sc_gather(table3, idx3, n_chunks, width):
    rows_per_w = n_chunks * CHUNK
    half = rows_per_w // 2
    total = NW * rows_per_w
    mesh = plsc.VectorSubcoreMesh(core_axis_name="c", subcore_axis_name="s")

    @functools.partial(
        pl.kernel,
        out_type=jax.ShapeDtypeStruct((total, width), jnp.int32),
        mesh=mesh,
        scratch_types=[
            pltpu.VMEM((n_chunks, CHUNK), jnp.int32),
            pltpu.VMEM((half, width), jnp.int32),
            pltpu.SemaphoreType.DMA,
        ],
        compiler_params=pltpu.CompilerParams(use_tc_tiling_on_sc=True),
    )
    def gather_kernel(table_hbm, idx_hbm, out_hbm, idx_v, rows_v, sem):
        wid = lax.axis_index("s") * NC + lax.axis_index("c")
        pltpu.sync_copy(idx_hbm.at[wid], idx_v)
        for h in range(2):
            copies = [
                pltpu.async_copy(
                    table_hbm.at[idx_v.at[h * (n_chunks // 2) + j]],
                    rows_v.at[pl.ds(j * CHUNK, CHUNK)],
                    sem,
                )
                for j in range(n_chunks // 2)
            ]
            for c in copies:
                c.wait()
            pltpu.sync_copy(rows_v, out_hbm.at[pl.ds(wid * rows_per_w + h * half, half)])

    return gather_kernel(table3, idx3)


def kernel(current_node, actionList, table, W, b):
    B = current_node.shape[0]
    vocab, embed = table.shape
    width = 2 * embed
    last_node = actionList[:, -2]
    idx = jnp.concatenate([current_node, last_node]).astype(jnp.int32)
    quad_row = ((idx >> MLOG) << (MLOG - 2)) | (idx & (Q - 1))
    idx2 = idx.reshape(2 * B, 1)

    n_chunks = (2 * B) // (NW * CHUNK)
    idx3 = quad_row.reshape(NW, n_chunks, CHUNK)

    table3 = _tc_relayout(table.T, vocab, embed)
    gathered = _sc_gather(table3, idx3, n_chunks, width)

    BM = 2048
    grid = B // BM
    w1 = W[:embed]
    w2 = W[embed:]
    b2 = b.reshape(1, embed)

    def pick(g, sl, hf):
        word = jnp.where(sl == 0, g & 0xFFFF, (g >> 16) & 0xFFFF)
        row = lax.bitcast_convert_type(word.astype(jnp.uint16), jnp.bfloat16)
        return jnp.where(hf == 0, row[:, :embed], row[:, embed:]).astype(jnp.float32)

    def proj(cur_ref, last_ref, scur_ref, slast_ref, hcur_ref, hlast_ref,
             w1_ref, w2_ref, b_ref, o_ref):
        cur_e = pick(cur_ref[...], scur_ref[...], hcur_ref[...])
        last_e = pick(last_ref[...], slast_ref[...], hlast_ref[...])
        res = (
            jnp.dot(cur_e, w1_ref[...], preferred_element_type=jnp.float32)
            + jnp.dot(last_e, w2_ref[...], preferred_element_type=jnp.float32)
            + b_ref[...]
        )
        o_ref[...] = res.T

    outT = pl.pallas_call(
        proj,
        grid=(grid,),
        in_specs=[
            pl.BlockSpec((BM, width), lambda i: (i, 0)),
            pl.BlockSpec((BM, width), lambda i: (i + grid, 0)),
            pl.BlockSpec((BM, 1), lambda i: (i, 0)),
            pl.BlockSpec((BM, 1), lambda i: (i + grid, 0)),
            pl.BlockSpec((embed, embed), lambda i: (0, 0)),
            pl.BlockSpec((embed, embed), lambda i: (0, 0)),
            pl.BlockSpec((1, embed), lambda i: (0, 0)),
        ],
        out_specs=pl.BlockSpec((embed, BM), lambda i: (0, i)),
        out_shape=jax.ShapeDtypeStruct((embed, B), jnp.float32),
    )(gathered, gathered, idx2, idx2, w1, w2, b2)
    return outT.T
